# baseline (speedup 1.0000x reference)
import dataclasses

import jax
import jax.numpy as jnp
import numpy as np
from jax import lax
from jax.experimental import pallas as pl
from jax.experimental.pallas import tpu as pltpu
from jax.experimental.pallas import tpu_sc as plsc

N = 10000
E = 160000
F_IN = 256
H_DIM = 256
C_DIM = 64
D = 128

NC = 2
NS = 16
LANES = 16
CH = 128
GS = 4

E_PAD = 163840
ZR = 624
ZR_LAST = 640


def _mm1_body(x_ref, w_ref, b_ref, o_ref):
    h = jnp.dot(x_ref[...], w_ref[...], preferred_element_type=jnp.float32)
    h = h + b_ref[...]
    o_ref[0] = h[:, :D]
    o_ref[1] = h[:, D:]


def _mm2_body(h_ref, w_ref, b_ref, o_ref):
    hl = jnp.maximum(h_ref[0], 0.0)
    hr = jnp.maximum(h_ref[1], 0.0)
    o_ref[...] = (
        jnp.dot(hl, w_ref[:D], preferred_element_type=jnp.float32)
        + jnp.dot(hr, w_ref[D:], preferred_element_type=jnp.float32)
        + b_ref[...]
    )


def _lsm_body(p_ref, o_ref):
    s = (p_ref[0] + p_ref[1])[:, :C_DIM]
    m = jnp.max(s, axis=1, keepdims=True)
    e = jnp.exp(s - m)
    lse = jnp.log(jnp.sum(e, axis=1, keepdims=True))
    o_ref[...] = s - m - lse


def _make_spmm_body(split_edges, n_scale_chunks):
    if split_edges:
        ept = E_PAD // (NC * NS)
    else:
        ept = E_PAD // NS
    nch = ept // CH
    half = nch // 2

    def body(h_hbm, src_hbm, dst_hbm, w_hbm, z_hbm, out_hbm,
             src_all, dst_v0, dst_v1, w_v0, w_v1, rows0, rows1, acc_sh,
             sg0a, sg0b, sg1a, sg1b, sa0, sa1, ss0, ss1):
        sg0 = (sg0a, sg0b)
        sg1 = (sg1a, sg1b)
        c = lax.axis_index("c")
        s = lax.axis_index("s")

        @pl.when(s < jnp.int32(NS - 1))
        def _():
            pltpu.sync_copy(z_hbm.at[pl.ds(0, ZR)],
                            acc_sh.at[pl.ds(s * jnp.int32(ZR), ZR)])

        @pl.when(s == jnp.int32(NS - 1))
        def _():
            pltpu.sync_copy(z_hbm, acc_sh.at[pl.ds(jnp.int32(ZR * (NS - 1)), ZR_LAST)])

        if split_edges:
            tile = s * jnp.int32(NC) + c
        else:
            tile = s
        ebase = tile * jnp.int32(ept)

        pltpu.sync_copy(src_hbm.at[pl.ds(ebase, ept)], src_all)
        plsc.subcore_barrier()

        def issue(k, dst_v, w_v, rows, sg, sa):
            off = pl.ds(ebase + k * jnp.int32(CH), CH)
            pltpu.async_copy(dst_hbm.at[off], dst_v, sa)
            pltpu.async_copy(w_hbm.at[off], w_v, sa)
            for sub in range(GS):
                sc_ = CH // GS
                idx = src_all.at[pl.ds(k * jnp.int32(CH) + jnp.int32(sub * sc_), sc_)]
                rsub = rows.at[pl.ds(sub * sc_, sc_)]
                if split_edges:
                    pltpu.async_copy(h_hbm.at[idx], rsub, sg[sub % 2])
                else:
                    pltpu.async_copy(h_hbm.at[c].at[idx], rsub, sg[sub % 2])

        def wait(k, dst_v, w_v, rows, sg, sa):
            off = pl.ds(ebase + k * jnp.int32(CH), CH)
            pltpu.make_async_copy(dst_hbm.at[off], dst_v, sa).wait()
            pltpu.make_async_copy(w_hbm.at[off], w_v, sa).wait()
            for sub in range(GS):
                sc_ = CH // GS
                idx = src_all.at[pl.ds(k * jnp.int32(CH) + jnp.int32(sub * sc_), sc_)]
                rsub = rows.at[pl.ds(sub * sc_, sc_)]
                if split_edges:
                    pltpu.make_async_copy(h_hbm.at[idx], rsub, sg[sub % 2]).wait()
                else:
                    pltpu.make_async_copy(h_hbm.at[c].at[idx], rsub, sg[sub % 2]).wait()

        def scale(w_v, rows):
            @plsc.parallel_loop(jnp.int32(0), jnp.int32(CH), step=jnp.int32(1), unroll=4)
            def _(e):
                eidx = jnp.full((LANES,), e, jnp.int32)
                wv = plsc.load_gather(w_v, [eidx])
                for j in range(n_scale_chunks):
                    sl = (e, pl.ds(j * LANES, LANES))
                    rows[sl] = rows[sl] * wv

        def scatter_start(dst_v, rows, ss):
            pltpu.async_copy(rows, acc_sh.at[dst_v], ss, add=True)

        def scatter_wait(dst_v, rows, ss):
            pltpu.make_async_copy(rows, acc_sh.at[dst_v], ss).wait()

        issue(jnp.int32(0), dst_v0, w_v0, rows0, sg0, sa0)

        @pl.loop(jnp.int32(0), jnp.int32(half))
        def _(k2):
            k0 = k2 * jnp.int32(2)
            k1 = k0 + jnp.int32(1)
            issue(k1, dst_v1, w_v1, rows1, sg1, sa1)
            wait(k0, dst_v0, w_v0, rows0, sg0, sa0)
            scale(w_v0, rows0)
            scatter_start(dst_v0, rows0, ss0)

            @pl.when(k2 + jnp.int32(1) < jnp.int32(half))
            def _():
                scatter_wait(dst_v0, rows0, ss0)
                issue(k0 + jnp.int32(2), dst_v0, w_v0, rows0, sg0, sa0)

            wait(k1, dst_v1, w_v1, rows1, sg1, sa1)
            scale(w_v1, rows1)
            scatter_start(dst_v1, rows1, ss1)

            @pl.when(k2 + jnp.int32(1) < jnp.int32(half))
            def _():
                scatter_wait(dst_v1, rows1, ss1)

        scatter_wait(dst_v0, rows0, ss0)
        scatter_wait(dst_v1, rows1, ss1)
        plsc.subcore_barrier()

        @pl.when(s < jnp.int32(NS - 1))
        def _():
            rs = pl.ds(s * jnp.int32(ZR), ZR)
            pltpu.sync_copy(acc_sh.at[rs], out_hbm.at[c].at[rs])

        @pl.when(s == jnp.int32(NS - 1))
        def _():
            rs = pl.ds(jnp.int32(ZR * (NS - 1)), ZR_LAST)
            pltpu.sync_copy(acc_sh.at[rs], out_hbm.at[c].at[rs])

    return body


_MESH = plsc.VectorSubcoreMesh(core_axis_name="c", subcore_axis_name="s")

_CP = pltpu.CompilerParams()
if "needs_layout_passes" in pltpu.CompilerParams.__dataclass_fields__:
    _CP = dataclasses.replace(_CP, needs_layout_passes=False)

def _sc_scratch(ept):
    return [
        pltpu.VMEM((ept,), jnp.int32),
        pltpu.VMEM((CH,), jnp.int32),
        pltpu.VMEM((CH,), jnp.int32),
        pltpu.VMEM((CH,), jnp.float32),
        pltpu.VMEM((CH,), jnp.float32),
        pltpu.VMEM((CH, D), jnp.float32),
        pltpu.VMEM((CH, D), jnp.float32),
        pltpu.VMEM_SHARED((N, D), jnp.float32),
        pltpu.SemaphoreType.DMA,
        pltpu.SemaphoreType.DMA,
        pltpu.SemaphoreType.DMA,
        pltpu.SemaphoreType.DMA,
        pltpu.SemaphoreType.DMA,
        pltpu.SemaphoreType.DMA,
        pltpu.SemaphoreType.DMA,
        pltpu.SemaphoreType.DMA,
    ]


_spmm1 = pl.kernel(
    _make_spmm_body(split_edges=False, n_scale_chunks=D // LANES),
    out_type=jax.ShapeDtypeStruct((NC, N, D), jnp.float32),
    mesh=_MESH,
    compiler_params=_CP,
    scratch_types=_sc_scratch(E_PAD // NS),
)

_spmm2 = pl.kernel(
    _make_spmm_body(split_edges=True, n_scale_chunks=C_DIM // LANES),
    out_type=jax.ShapeDtypeStruct((NC, N, D), jnp.float32),
    mesh=_MESH,
    compiler_params=_CP,
    scratch_types=_sc_scratch(E_PAD // (NC * NS)),
)

_BR = 1000
_I0 = np.int32(0)

_mm1 = pl.pallas_call(
    _mm1_body,
    grid=(N // _BR,),
    in_specs=[
        pl.BlockSpec((_BR, F_IN), lambda i: (i, _I0)),
        pl.BlockSpec((F_IN, H_DIM), lambda i: (_I0, _I0)),
        pl.BlockSpec((1, H_DIM), lambda i: (_I0, _I0)),
    ],
    out_specs=pl.BlockSpec((NC, _BR, D), lambda i: (_I0, i, _I0)),
    out_shape=jax.ShapeDtypeStruct((NC, N, D), jnp.float32),
)

_mm2 = pl.pallas_call(
    _mm2_body,
    grid=(N // _BR,),
    in_specs=[
        pl.BlockSpec((NC, _BR, D), lambda i: (_I0, i, _I0)),
        pl.BlockSpec((H_DIM, D), lambda i: (_I0, _I0)),
        pl.BlockSpec((1, D), lambda i: (_I0, _I0)),
    ],
    out_specs=pl.BlockSpec((_BR, D), lambda i: (i, _I0)),
    out_shape=jax.ShapeDtypeStruct((N, D), jnp.float32),
)

_lsm = pl.pallas_call(
    _lsm_body,
    grid=(N // _BR,),
    in_specs=[pl.BlockSpec((NC, _BR, D), lambda i: (_I0, i, _I0))],
    out_specs=pl.BlockSpec((_BR, C_DIM), lambda i: (i, _I0)),
    out_shape=jax.ShapeDtypeStruct((N, C_DIM), jnp.float32),
)


@jax.jit
def kernel(x, edge_index, edge_weight, W1, b1, W2, b2):
    x = x.astype(jnp.float32)
    pad = E_PAD - E
    dst = jnp.pad(edge_index[0].astype(jnp.int32), (0, pad))
    src = jnp.pad(edge_index[1].astype(jnp.int32), (0, pad))
    w = jnp.pad(edge_weight.astype(jnp.float32), (0, pad))

    w1t = W1.astype(jnp.float32).T
    w2t = jnp.pad(W2.astype(jnp.float32).T, ((0, 0), (0, D - C_DIM)))
    b1r = b1.astype(jnp.float32).reshape(1, H_DIM)
    b2r = jnp.pad(b2.astype(jnp.float32), (0, D - C_DIM)).reshape(1, D)

    zrows = jnp.zeros((ZR_LAST, D), jnp.float32)

    h_halves = _mm1(x, w1t, b1r)
    h_agg = _spmm1(h_halves, src, dst, w, zrows)
    h2 = _mm2(h_agg, w2t, b2r)
    partial = _spmm2(h2, src, dst, w, zrows)
    return _lsm(partial)


# speedup vs baseline: 1.0019x; 1.0019x over previous
import dataclasses

import jax
import jax.numpy as jnp
import numpy as np
from jax import lax
from jax.experimental import pallas as pl
from jax.experimental.pallas import tpu as pltpu
from jax.experimental.pallas import tpu_sc as plsc

N = 10000
E = 160000
F_IN = 256
H_DIM = 256
C_DIM = 64
D = 128

NC = 2
NS = 16
LANES = 16
CH = 128

E_PAD = 163840
ZR = 624
ZR_LAST = 640


def _mm1_body(x_ref, w_ref, b_ref, o_ref):
    h = jnp.dot(x_ref[...], w_ref[...], preferred_element_type=jnp.float32)
    h = h + b_ref[...]
    o_ref[0] = h[:, :D]
    o_ref[1] = h[:, D:]


def _mm2_body(h_ref, w_ref, b_ref, o_ref):
    hl = jnp.maximum(h_ref[0], 0.0)
    hr = jnp.maximum(h_ref[1], 0.0)
    o_ref[...] = (
        jnp.dot(hl, w_ref[:D], preferred_element_type=jnp.float32)
        + jnp.dot(hr, w_ref[D:], preferred_element_type=jnp.float32)
        + b_ref[...]
    )


def _lsm_body(p_ref, o_ref):
    s = (p_ref[0] + p_ref[1])[:, :C_DIM]
    m = jnp.max(s, axis=1, keepdims=True)
    e = jnp.exp(s - m)
    lse = jnp.log(jnp.sum(e, axis=1, keepdims=True))
    o_ref[...] = s - m - lse


def _make_spmm_body(split_edges, n_scale_chunks):
    if split_edges:
        ept = E_PAD // (NC * NS)
    else:
        ept = E_PAD // NS
    nch = ept // CH
    half = nch // 2

    def body(h_hbm, src_hbm, dst_hbm, w_hbm, z_hbm, out_hbm,
             src_all, dst_v0, dst_v1, w_v0, w_v1, rows0, rows1, acc_sh,
             sg0, sg1, sa0, sa1, ss0, ss1):
        c = lax.axis_index("c")
        s = lax.axis_index("s")

        @pl.when(s < jnp.int32(NS - 1))
        def _():
            pltpu.sync_copy(z_hbm.at[pl.ds(0, ZR)],
                            acc_sh.at[pl.ds(s * jnp.int32(ZR), ZR)])

        @pl.when(s == jnp.int32(NS - 1))
        def _():
            pltpu.sync_copy(z_hbm, acc_sh.at[pl.ds(jnp.int32(ZR * (NS - 1)), ZR_LAST)])

        if split_edges:
            tile = s * jnp.int32(NC) + c
        else:
            tile = s
        ebase = tile * jnp.int32(ept)

        pltpu.sync_copy(src_hbm.at[pl.ds(ebase, ept)], src_all)
        plsc.subcore_barrier()

        def issue(k, dst_v, w_v, rows, sg, sa):
            off = pl.ds(ebase + k * jnp.int32(CH), CH)
            pltpu.async_copy(dst_hbm.at[off], dst_v, sa)
            pltpu.async_copy(w_hbm.at[off], w_v, sa)
            idx = src_all.at[pl.ds(k * jnp.int32(CH), CH)]
            if split_edges:
                pltpu.async_copy(h_hbm.at[idx], rows, sg)
            else:
                pltpu.async_copy(h_hbm.at[c].at[idx], rows, sg)

        def wait(k, dst_v, w_v, rows, sg, sa):
            off = pl.ds(ebase + k * jnp.int32(CH), CH)
            pltpu.make_async_copy(dst_hbm.at[off], dst_v, sa).wait()
            pltpu.make_async_copy(w_hbm.at[off], w_v, sa).wait()
            idx = src_all.at[pl.ds(k * jnp.int32(CH), CH)]
            if split_edges:
                pltpu.make_async_copy(h_hbm.at[idx], rows, sg).wait()
            else:
                pltpu.make_async_copy(h_hbm.at[c].at[idx], rows, sg).wait()

        def scale(w_v, rows):
            @plsc.parallel_loop(jnp.int32(0), jnp.int32(CH), step=jnp.int32(1), unroll=4)
            def _(e):
                eidx = jnp.full((LANES,), e, jnp.int32)
                wv = plsc.load_gather(w_v, [eidx])
                for j in range(n_scale_chunks):
                    sl = (e, pl.ds(j * LANES, LANES))
                    rows[sl] = rows[sl] * wv

        def scatter_start(dst_v, rows, ss):
            pltpu.async_copy(rows, acc_sh.at[dst_v], ss, add=True)

        def scatter_wait(dst_v, rows, ss):
            pltpu.make_async_copy(rows, acc_sh.at[dst_v], ss).wait()

        issue(jnp.int32(0), dst_v0, w_v0, rows0, sg0, sa0)

        @pl.loop(jnp.int32(0), jnp.int32(half))
        def _(k2):
            k0 = k2 * jnp.int32(2)
            k1 = k0 + jnp.int32(1)
            issue(k1, dst_v1, w_v1, rows1, sg1, sa1)
            wait(k0, dst_v0, w_v0, rows0, sg0, sa0)
            scale(w_v0, rows0)
            scatter_start(dst_v0, rows0, ss0)

            @pl.when(k2 + jnp.int32(1) < jnp.int32(half))
            def _():
                scatter_wait(dst_v0, rows0, ss0)
                issue(k0 + jnp.int32(2), dst_v0, w_v0, rows0, sg0, sa0)

            wait(k1, dst_v1, w_v1, rows1, sg1, sa1)
            scale(w_v1, rows1)
            scatter_start(dst_v1, rows1, ss1)

            @pl.when(k2 + jnp.int32(1) < jnp.int32(half))
            def _():
                scatter_wait(dst_v1, rows1, ss1)

        scatter_wait(dst_v0, rows0, ss0)
        scatter_wait(dst_v1, rows1, ss1)
        plsc.subcore_barrier()

        @pl.when(s < jnp.int32(NS - 1))
        def _():
            rs = pl.ds(s * jnp.int32(ZR), ZR)
            pltpu.sync_copy(acc_sh.at[rs], out_hbm.at[c].at[rs])

        @pl.when(s == jnp.int32(NS - 1))
        def _():
            rs = pl.ds(jnp.int32(ZR * (NS - 1)), ZR_LAST)
            pltpu.sync_copy(acc_sh.at[rs], out_hbm.at[c].at[rs])

    return body


_MESH = plsc.VectorSubcoreMesh(core_axis_name="c", subcore_axis_name="s")

_CP = pltpu.CompilerParams()
if "needs_layout_passes" in pltpu.CompilerParams.__dataclass_fields__:
    _CP = dataclasses.replace(_CP, needs_layout_passes=False)

def _sc_scratch(ept):
    return [
        pltpu.VMEM((ept,), jnp.int32),
        pltpu.VMEM((CH,), jnp.int32),
        pltpu.VMEM((CH,), jnp.int32),
        pltpu.VMEM((CH,), jnp.float32),
        pltpu.VMEM((CH,), jnp.float32),
        pltpu.VMEM((CH, D), jnp.float32),
        pltpu.VMEM((CH, D), jnp.float32),
        pltpu.VMEM_SHARED((N, D), jnp.float32),
        pltpu.SemaphoreType.DMA,
        pltpu.SemaphoreType.DMA,
        pltpu.SemaphoreType.DMA,
        pltpu.SemaphoreType.DMA,
        pltpu.SemaphoreType.DMA,
        pltpu.SemaphoreType.DMA,
    ]


_spmm1 = pl.kernel(
    _make_spmm_body(split_edges=False, n_scale_chunks=D // LANES),
    out_type=jax.ShapeDtypeStruct((NC, N, D), jnp.float32),
    mesh=_MESH,
    compiler_params=_CP,
    scratch_types=_sc_scratch(E_PAD // NS),
)

_spmm2 = pl.kernel(
    _make_spmm_body(split_edges=True, n_scale_chunks=C_DIM // LANES),
    out_type=jax.ShapeDtypeStruct((NC, N, D), jnp.float32),
    mesh=_MESH,
    compiler_params=_CP,
    scratch_types=_sc_scratch(E_PAD // (NC * NS)),
)

_BR = 1000
_I0 = np.int32(0)

_mm1 = pl.pallas_call(
    _mm1_body,
    grid=(N // _BR,),
    in_specs=[
        pl.BlockSpec((_BR, F_IN), lambda i: (i, _I0)),
        pl.BlockSpec((F_IN, H_DIM), lambda i: (_I0, _I0)),
        pl.BlockSpec((1, H_DIM), lambda i: (_I0, _I0)),
    ],
    out_specs=pl.BlockSpec((NC, _BR, D), lambda i: (_I0, i, _I0)),
    out_shape=jax.ShapeDtypeStruct((NC, N, D), jnp.float32),
)

_mm2 = pl.pallas_call(
    _mm2_body,
    grid=(N // _BR,),
    in_specs=[
        pl.BlockSpec((NC, _BR, D), lambda i: (_I0, i, _I0)),
        pl.BlockSpec((H_DIM, D), lambda i: (_I0, _I0)),
        pl.BlockSpec((1, D), lambda i: (_I0, _I0)),
    ],
    out_specs=pl.BlockSpec((_BR, D), lambda i: (i, _I0)),
    out_shape=jax.ShapeDtypeStruct((N, D), jnp.float32),
)

_lsm = pl.pallas_call(
    _lsm_body,
    grid=(N // _BR,),
    in_specs=[pl.BlockSpec((NC, _BR, D), lambda i: (_I0, i, _I0))],
    out_specs=pl.BlockSpec((_BR, C_DIM), lambda i: (i, _I0)),
    out_shape=jax.ShapeDtypeStruct((N, C_DIM), jnp.float32),
)


@jax.jit
def kernel(x, edge_index, edge_weight, W1, b1, W2, b2):
    x = x.astype(jnp.float32)
    pad = E_PAD - E
    dst = jnp.pad(edge_index[0].astype(jnp.int32), (0, pad))
    src = jnp.pad(edge_index[1].astype(jnp.int32), (0, pad))
    w = jnp.pad(edge_weight.astype(jnp.float32), (0, pad))

    w1t = W1.astype(jnp.float32).T
    w2t = jnp.pad(W2.astype(jnp.float32).T, ((0, 0), (0, D - C_DIM)))
    b1r = b1.astype(jnp.float32).reshape(1, H_DIM)
    b2r = jnp.pad(b2.astype(jnp.float32), (0, D - C_DIM)).reshape(1, D)

    zrows = jnp.zeros((ZR_LAST, D), jnp.float32)

    h_halves = _mm1(x, w1t, b1r)
    h_agg = _spmm1(h_halves, src, dst, w, zrows)
    h2 = _mm2(h_agg, w2t, b2r)
    partial = _spmm2(h2, src, dst, w, zrows)
    return _lsm(partial)
